# Initial kernel scaffold; baseline (speedup 1.0000x reference)
import jax
import jax.numpy as jnp
from jax.experimental import pallas as pl


def kernel(x, edge_index, W1_l, b1, W1_r, W2_l, b2, W2_r):
    raise NotImplementedError("write your pallas kernel here")


# speedup vs baseline: 5.2789x; 5.2789x over previous
import functools

import jax
import jax.numpy as jnp
from jax import lax
from jax.experimental import pallas as pl
from jax.experimental.pallas import tpu as pltpu
from jax.experimental.pallas import tpu_sc as plsc

N = 10000
C = 128
E = 320000
NC = 2
NS = 16
NW = NC * NS
EPW = E // NW
CH = 80
NCHUNK = EPW // CH
NP = 10240
RPT = NP // NS
OW = 16
BLK = 400

_mesh = plsc.VectorSubcoreMesh(
    core_axis_name="c", subcore_axis_name="s", num_cores=NC, num_subcores=NS
)


def _seg_body(with_cnt, x_hbm, src_hbm, dst_hbm, zacc_hbm, zcnt_hbm, ones_hbm,
              out_hbm, cnt_hbm, sidx_v, didx_v, rows_v, ones_v, cnt_v, acc_sh,
              cnt_sh, sem):
    cid = lax.axis_index("c")
    sid = lax.axis_index("s")

    pltpu.sync_copy(zacc_hbm, rows_v)
    if with_cnt:
        pltpu.sync_copy(ones_hbm, ones_v)
        pltpu.sync_copy(zcnt_hbm, cnt_v)
        pltpu.sync_copy(cnt_v, cnt_sh.at[pl.ds(sid * RPT, RPT)])
    for j in range(RPT // CH):
        off = sid * RPT + j * CH
        pltpu.sync_copy(rows_v, acc_sh.at[pl.ds(off, CH)])
    plsc.subcore_barrier()

    ebase = (cid * NS + sid) * EPW

    def body(k, carry):
        off = ebase + k * CH
        pltpu.sync_copy(src_hbm.at[pl.ds(off, CH)], sidx_v)
        pltpu.sync_copy(dst_hbm.at[pl.ds(off, CH)], didx_v)
        pltpu.async_copy(x_hbm.at[sidx_v], rows_v, sem).wait()
        pltpu.sync_copy(rows_v, acc_sh.at[didx_v], add=True)
        if with_cnt:
            pltpu.sync_copy(ones_v, cnt_sh.at[didx_v], add=True)
        return carry

    lax.fori_loop(0, NCHUNK, body, 0)
    plsc.subcore_barrier()

    for j in range(RPT // CH):
        off = sid * RPT + j * CH
        pltpu.sync_copy(acc_sh.at[pl.ds(off, CH)], rows_v)
        pltpu.sync_copy(rows_v, out_hbm.at[cid, pl.ds(off, CH)])
    if with_cnt:
        pltpu.sync_copy(cnt_sh.at[pl.ds(sid * RPT, RPT)], cnt_v)
        pltpu.sync_copy(cnt_v, cnt_hbm.at[pl.ds(cid * NP + sid * RPT, RPT)])


_seg_sum_cnt = functools.partial(
    pl.kernel,
    out_type=[
        jax.ShapeDtypeStruct((NC, NP, C), jnp.float32),
        jax.ShapeDtypeStruct((NC * NP,), jnp.float32),
    ],
    mesh=_mesh,
    scratch_types=[
        pltpu.VMEM((CH,), jnp.int32),
        pltpu.VMEM((CH,), jnp.int32),
        pltpu.VMEM((CH, C), jnp.float32),
        pltpu.VMEM((CH,), jnp.float32),
        pltpu.VMEM((RPT,), jnp.float32),
        pltpu.VMEM_SHARED((NP, C), jnp.float32),
        pltpu.VMEM_SHARED((NP,), jnp.float32),
        pltpu.SemaphoreType.DMA,
    ],
)(functools.partial(_seg_body, True))


def _seg_nocnt_body(x_hbm, src_hbm, dst_hbm, zacc_hbm, out_hbm, sidx_v, didx_v,
                    rows_v, acc_sh, sem):
    _seg_body(False, x_hbm, src_hbm, dst_hbm, zacc_hbm, None, None, out_hbm,
              None, sidx_v, didx_v, rows_v, None, None, acc_sh, None, sem)


_seg_sum = functools.partial(
    pl.kernel,
    out_type=jax.ShapeDtypeStruct((NC, NP, C), jnp.float32),
    mesh=_mesh,
    scratch_types=[
        pltpu.VMEM((CH,), jnp.int32),
        pltpu.VMEM((CH,), jnp.int32),
        pltpu.VMEM((CH, C), jnp.float32),
        pltpu.VMEM_SHARED((NP, C), jnp.float32),
        pltpu.SemaphoreType.DMA,
    ],
)(_seg_nocnt_body)


def _combine_body(relu, parts_ref, cnt_ref, x_ref, wl_ref, b_ref, wr_ref, o_ref):
    agg = parts_ref[0] + parts_ref[1]
    cnt = cnt_ref[0] + cnt_ref[1]
    mean = agg / jnp.maximum(cnt, 1.0)
    out = (
        lax.dot_general(mean, wl_ref[...], (((1,), (1,)), ((), ())),
                        preferred_element_type=jnp.float32)
        + lax.dot_general(x_ref[...], wr_ref[...], (((1,), (1,)), ((), ())),
                          preferred_element_type=jnp.float32)
        + b_ref[0:1, :]
    )
    if relu:
        out = jnp.maximum(out, 0.0)
    o_ref[...] = out


def _combine(parts, cnt1, x, w_l, b, w_r, relu):
    b8 = jnp.broadcast_to(b.reshape(1, C), (8, C))
    return pl.pallas_call(
        functools.partial(_combine_body, relu),
        grid=(N // BLK,),
        in_specs=[
            pl.BlockSpec((NC, BLK, C), lambda i: (0, i, 0)),
            pl.BlockSpec((NC, BLK, 1), lambda i: (0, i, 0)),
            pl.BlockSpec((BLK, C), lambda i: (i, 0)),
            pl.BlockSpec((C, C), lambda i: (0, 0)),
            pl.BlockSpec((8, C), lambda i: (0, 0)),
            pl.BlockSpec((C, C), lambda i: (0, 0)),
        ],
        out_specs=pl.BlockSpec((BLK, C), lambda i: (i, 0)),
        out_shape=jax.ShapeDtypeStruct((N, C), jnp.float32),
    )(parts, cnt1, x, w_l, b8, w_r)


def kernel(x, edge_index, W1_l, b1, W1_r, W2_l, b2, W2_r):
    ei = edge_index.astype(jnp.int32)
    src, dst = ei[0], ei[1]
    zacc = jnp.zeros((CH, C), jnp.float32)
    zcnt = jnp.zeros((RPT,), jnp.float32)
    ones = jnp.ones((CH,), jnp.float32)

    parts1, cntp = _seg_sum_cnt(x, src, dst, zacc, zcnt, ones)
    parts1 = parts1[:, :N]
    cnt1 = cntp.reshape(NC, NP)[:, :N, None]
    h = _combine(parts1, cnt1, x, W1_l, b1, W1_r, relu=True)
    parts2 = _seg_sum(h, src, dst, zacc)[:, :N]
    out = _combine(parts2, cnt1, h, W2_l, b2, W2_r, relu=False)
    return out


# speedup vs baseline: 9.7659x; 1.8500x over previous
import functools

import jax
import jax.numpy as jnp
from jax import lax
from jax.experimental import pallas as pl
from jax.experimental.pallas import tpu as pltpu
from jax.experimental.pallas import tpu_sc as plsc

N = 10000
C = 128
E = 320000
NC = 2
NS = 16
NW = NC * NS
EPW = E // NW
CH = 80
NCHUNK = EPW // CH
NPAIR = (NCHUNK - 1) // 2
NP = 10240
RPT = NP // NS
BLK = 400

_mesh = plsc.VectorSubcoreMesh(
    core_axis_name="c", subcore_axis_name="s", num_cores=NC, num_subcores=NS
)


def _seg_body(with_cnt, x_hbm, src_hbm, dst_hbm, zacc_hbm, zcnt_hbm, ones_hbm,
              out_hbm, cnt_hbm, sidx0, sidx1, didx0, didx1, rows0, rows1,
              ones_v, cnt_v, acc_sh, cnt_sh, gs0, gs1, is0, is1, id0, id1,
              osem):
    cid = lax.axis_index("c")
    sid = lax.axis_index("s")
    sidx = (sidx0, sidx1)
    didx = (didx0, didx1)
    rows = (rows0, rows1)
    gs = (gs0, gs1)
    iss = (is0, is1)
    ids = (id0, id1)

    pltpu.sync_copy(zacc_hbm, rows0)
    if with_cnt:
        pltpu.sync_copy(ones_hbm, ones_v)
        pltpu.sync_copy(zcnt_hbm, cnt_v)
        pltpu.sync_copy(cnt_v, cnt_sh.at[pl.ds(sid * RPT, RPT)])
    for j in range(RPT // CH):
        pltpu.sync_copy(rows0, acc_sh.at[pl.ds(sid * RPT + j * CH, CH)])
    plsc.subcore_barrier()

    ebase = (cid * NS + sid) * EPW

    def off(c):
        return ebase + jnp.minimum(c, NCHUNK - 1) * CH

    def idx_load(c, b):
        pltpu.async_copy(src_hbm.at[pl.ds(off(c), CH)], sidx[b], iss[b])
        pltpu.async_copy(dst_hbm.at[pl.ds(off(c), CH)], didx[b], ids[b])

    def idx_wait(b):
        pltpu.make_async_copy(src_hbm.at[pl.ds(0, CH)], sidx[b], iss[b]).wait()
        pltpu.make_async_copy(dst_hbm.at[pl.ds(0, CH)], didx[b], ids[b]).wait()

    def gather(b):
        pltpu.async_copy(x_hbm.at[sidx[b]], rows[b], gs[b])

    def gather_wait(b):
        pltpu.make_async_copy(x_hbm.at[sidx[b]], rows[b], gs[b]).wait()

    def scatter(b):
        if with_cnt:
            pltpu.async_copy(ones_v, cnt_sh.at[didx[b]], osem, add=True)
        pltpu.sync_copy(rows[b], acc_sh.at[didx[b]], add=True)
        if with_cnt:
            pltpu.make_async_copy(ones_v, cnt_sh.at[didx[b]], osem).wait()

    pltpu.sync_copy(src_hbm.at[pl.ds(ebase, CH)], sidx0)
    pltpu.sync_copy(dst_hbm.at[pl.ds(ebase, CH)], didx0)
    idx_load(1, 1)
    gather(0)

    def body(i, carry):
        for b in (0, 1):
            c = 2 * i + b
            idx_wait(1 - b)
            gather(1 - b)
            gather_wait(b)
            scatter(b)
            idx_load(c + 2, b)
        return carry

    lax.fori_loop(0, NPAIR, body, 0)

    gather_wait(0)
    scatter(0)
    idx_wait(1)
    plsc.subcore_barrier()

    for j in range(RPT // CH):
        o = sid * RPT + j * CH
        pltpu.sync_copy(acc_sh.at[pl.ds(o, CH)], rows0)
        pltpu.sync_copy(rows0, out_hbm.at[cid, pl.ds(o, CH)])
    if with_cnt:
        pltpu.sync_copy(cnt_sh.at[pl.ds(sid * RPT, RPT)], cnt_v)
        pltpu.sync_copy(cnt_v, cnt_hbm.at[pl.ds(cid * NP + sid * RPT, RPT)])


_seg_sum_cnt = functools.partial(
    pl.kernel,
    out_type=[
        jax.ShapeDtypeStruct((NC, NP, C), jnp.float32),
        jax.ShapeDtypeStruct((NC * NP,), jnp.float32),
    ],
    mesh=_mesh,
    scratch_types=[
        pltpu.VMEM((CH,), jnp.int32),
        pltpu.VMEM((CH,), jnp.int32),
        pltpu.VMEM((CH,), jnp.int32),
        pltpu.VMEM((CH,), jnp.int32),
        pltpu.VMEM((CH, C), jnp.float32),
        pltpu.VMEM((CH, C), jnp.float32),
        pltpu.VMEM((CH,), jnp.float32),
        pltpu.VMEM((RPT,), jnp.float32),
        pltpu.VMEM_SHARED((NP, C), jnp.float32),
        pltpu.VMEM_SHARED((NP,), jnp.float32),
        pltpu.SemaphoreType.DMA,
        pltpu.SemaphoreType.DMA,
        pltpu.SemaphoreType.DMA,
        pltpu.SemaphoreType.DMA,
        pltpu.SemaphoreType.DMA,
        pltpu.SemaphoreType.DMA,
        pltpu.SemaphoreType.DMA,
    ],
)(functools.partial(_seg_body, True))


def _seg_nocnt_body(x_hbm, src_hbm, dst_hbm, zacc_hbm, out_hbm, sidx0, sidx1,
                    didx0, didx1, rows0, rows1, acc_sh, gs0, gs1, is0, is1,
                    id0, id1):
    _seg_body(False, x_hbm, src_hbm, dst_hbm, zacc_hbm, None, None, out_hbm,
              None, sidx0, sidx1, didx0, didx1, rows0, rows1, None, None,
              acc_sh, None, gs0, gs1, is0, is1, id0, id1, None)


_seg_sum = functools.partial(
    pl.kernel,
    out_type=jax.ShapeDtypeStruct((NC, NP, C), jnp.float32),
    mesh=_mesh,
    scratch_types=[
        pltpu.VMEM((CH,), jnp.int32),
        pltpu.VMEM((CH,), jnp.int32),
        pltpu.VMEM((CH,), jnp.int32),
        pltpu.VMEM((CH,), jnp.int32),
        pltpu.VMEM((CH, C), jnp.float32),
        pltpu.VMEM((CH, C), jnp.float32),
        pltpu.VMEM_SHARED((NP, C), jnp.float32),
        pltpu.SemaphoreType.DMA,
        pltpu.SemaphoreType.DMA,
        pltpu.SemaphoreType.DMA,
        pltpu.SemaphoreType.DMA,
        pltpu.SemaphoreType.DMA,
        pltpu.SemaphoreType.DMA,
    ],
)(_seg_nocnt_body)


def _combine_body(relu, parts_ref, cnt_ref, x_ref, wl_ref, b_ref, wr_ref, o_ref):
    agg = parts_ref[0] + parts_ref[1]
    cnt = cnt_ref[0] + cnt_ref[1]
    mean = agg / jnp.maximum(cnt, 1.0)
    out = (
        lax.dot_general(mean, wl_ref[...], (((1,), (1,)), ((), ())),
                        preferred_element_type=jnp.float32)
        + lax.dot_general(x_ref[...], wr_ref[...], (((1,), (1,)), ((), ())),
                          preferred_element_type=jnp.float32)
        + b_ref[0:1, :]
    )
    if relu:
        out = jnp.maximum(out, 0.0)
    o_ref[...] = out


def _combine(parts, cnt1, x, w_l, b, w_r, relu):
    b8 = jnp.broadcast_to(b.reshape(1, C), (8, C))
    return pl.pallas_call(
        functools.partial(_combine_body, relu),
        grid=(N // BLK,),
        in_specs=[
            pl.BlockSpec((NC, BLK, C), lambda i: (0, i, 0)),
            pl.BlockSpec((NC, BLK, 1), lambda i: (0, i, 0)),
            pl.BlockSpec((BLK, C), lambda i: (i, 0)),
            pl.BlockSpec((C, C), lambda i: (0, 0)),
            pl.BlockSpec((8, C), lambda i: (0, 0)),
            pl.BlockSpec((C, C), lambda i: (0, 0)),
        ],
        out_specs=pl.BlockSpec((BLK, C), lambda i: (i, 0)),
        out_shape=jax.ShapeDtypeStruct((N, C), jnp.float32),
    )(parts, cnt1, x, w_l, b8, w_r)


def kernel(x, edge_index, W1_l, b1, W1_r, W2_l, b2, W2_r):
    ei = edge_index.astype(jnp.int32)
    src, dst = ei[0], ei[1]
    zacc = jnp.zeros((CH, C), jnp.float32)
    zcnt = jnp.zeros((RPT,), jnp.float32)
    ones = jnp.ones((CH,), jnp.float32)

    parts1, cntp = _seg_sum_cnt(x, src, dst, zacc, zcnt, ones)
    parts1 = parts1[:, :N]
    cnt1 = cntp.reshape(NC, NP)[:, :N, None]
    h = _combine(parts1, cnt1, x, W1_l, b1, W1_r, relu=True)
    parts2 = _seg_sum(h, src, dst, zacc)[:, :N]
    out = _combine(parts2, cnt1, h, W2_l, b2, W2_r, relu=False)
    return out


# speedup vs baseline: 9.9890x; 1.0228x over previous
import functools

import jax
import jax.numpy as jnp
from jax import lax
from jax.experimental import pallas as pl
from jax.experimental.pallas import tpu as pltpu
from jax.experimental.pallas import tpu_sc as plsc

N = 10000
C = 128
E = 320000
NC = 2
NS = 16
NW = NC * NS
EPW = E // NW
CH = 80
NCHUNK = EPW // CH
NPAIR = (NCHUNK - 1) // 2
NP = 10240
RPT = NP // NS
BLK = 400

_mesh = plsc.VectorSubcoreMesh(
    core_axis_name="c", subcore_axis_name="s", num_cores=NC, num_subcores=NS
)


def _seg_body(with_cnt, x_hbm, src_hbm, dst_hbm, zacc_hbm, zcnt_hbm, ones_hbm,
              out_hbm, cnt_hbm, sidx0, sidx1, didx0, didx1, rows0, rows1,
              ones_v, cnt_v, acc_sh, cnt_sh, gs0, gs1, is0, is1, id0, id1,
              osem):
    cid = lax.axis_index("c")
    sid = lax.axis_index("s")
    sidx = (sidx0, sidx1)
    didx = (didx0, didx1)
    rows = (rows0, rows1)
    gs = (gs0, gs1)
    iss = (is0, is1)
    ids = (id0, id1)

    pltpu.sync_copy(zacc_hbm, rows0)
    if with_cnt:
        pltpu.sync_copy(ones_hbm, ones_v)
        pltpu.sync_copy(zcnt_hbm, cnt_v)
        pltpu.sync_copy(cnt_v, cnt_sh.at[pl.ds(sid * RPT, RPT)])
    for j in range(RPT // CH):
        pltpu.sync_copy(rows0, acc_sh.at[pl.ds(sid * RPT + j * CH, CH)])
    plsc.subcore_barrier()

    ebase = (cid * NS + sid) * EPW

    def off(c):
        return ebase + jnp.minimum(c, NCHUNK - 1) * CH

    def idx_load(c, b):
        pltpu.async_copy(src_hbm.at[pl.ds(off(c), CH)], sidx[b], iss[b])
        pltpu.async_copy(dst_hbm.at[pl.ds(off(c), CH)], didx[b], ids[b])

    def idx_wait(b):
        pltpu.make_async_copy(src_hbm.at[pl.ds(0, CH)], sidx[b], iss[b]).wait()
        pltpu.make_async_copy(dst_hbm.at[pl.ds(0, CH)], didx[b], ids[b]).wait()

    def gather(b):
        pltpu.async_copy(x_hbm.at[sidx[b]], rows[b], gs[b])

    def gather_wait(b):
        pltpu.make_async_copy(x_hbm.at[sidx[b]], rows[b], gs[b]).wait()

    def scatter(b):
        if with_cnt:
            pltpu.async_copy(ones_v, cnt_sh.at[didx[b]], osem, add=True)
        pltpu.sync_copy(rows[b], acc_sh.at[didx[b]], add=True)
        if with_cnt:
            pltpu.make_async_copy(ones_v, cnt_sh.at[didx[b]], osem).wait()

    pltpu.sync_copy(src_hbm.at[pl.ds(ebase, CH)], sidx0)
    pltpu.sync_copy(dst_hbm.at[pl.ds(ebase, CH)], didx0)
    idx_load(1, 1)
    gather(0)

    def body(i, carry):
        for b in (0, 1):
            c = 2 * i + b
            idx_wait(1 - b)
            gather(1 - b)
            gather_wait(b)
            scatter(b)
            idx_load(c + 2, b)
        return carry

    lax.fori_loop(0, NPAIR, body, 0)

    gather_wait(0)
    scatter(0)
    idx_wait(1)
    plsc.subcore_barrier()

    for j in range(RPT // CH):
        o = sid * RPT + j * CH
        pltpu.sync_copy(acc_sh.at[pl.ds(o, CH)], rows0)
        pltpu.sync_copy(rows0, out_hbm.at[cid, pl.ds(o, CH)])
    if with_cnt:
        pltpu.sync_copy(cnt_sh.at[pl.ds(sid * RPT, RPT)], cnt_v)
        pltpu.sync_copy(cnt_v, cnt_hbm.at[pl.ds(cid * NP + sid * RPT, RPT)])


_seg_sum_cnt = functools.partial(
    pl.kernel,
    out_type=[
        jax.ShapeDtypeStruct((NC, NP, C), jnp.float32),
        jax.ShapeDtypeStruct((NC * NP,), jnp.float32),
    ],
    mesh=_mesh,
    scratch_types=[
        pltpu.VMEM((CH,), jnp.int32),
        pltpu.VMEM((CH,), jnp.int32),
        pltpu.VMEM((CH,), jnp.int32),
        pltpu.VMEM((CH,), jnp.int32),
        pltpu.VMEM((CH, C), jnp.float32),
        pltpu.VMEM((CH, C), jnp.float32),
        pltpu.VMEM((CH,), jnp.float32),
        pltpu.VMEM((RPT,), jnp.float32),
        pltpu.VMEM_SHARED((NP, C), jnp.float32),
        pltpu.VMEM_SHARED((NP,), jnp.float32),
        pltpu.SemaphoreType.DMA,
        pltpu.SemaphoreType.DMA,
        pltpu.SemaphoreType.DMA,
        pltpu.SemaphoreType.DMA,
        pltpu.SemaphoreType.DMA,
        pltpu.SemaphoreType.DMA,
        pltpu.SemaphoreType.DMA,
    ],
)(functools.partial(_seg_body, True))


def _seg_nocnt_body(x_hbm, src_hbm, dst_hbm, zacc_hbm, out_hbm, sidx0, sidx1,
                    didx0, didx1, rows0, rows1, acc_sh, gs0, gs1, is0, is1,
                    id0, id1):
    _seg_body(False, x_hbm, src_hbm, dst_hbm, zacc_hbm, None, None, out_hbm,
              None, sidx0, sidx1, didx0, didx1, rows0, rows1, None, None,
              acc_sh, None, gs0, gs1, is0, is1, id0, id1, None)


_seg_sum = functools.partial(
    pl.kernel,
    out_type=jax.ShapeDtypeStruct((NC, NP, C), jnp.float32),
    mesh=_mesh,
    scratch_types=[
        pltpu.VMEM((CH,), jnp.int32),
        pltpu.VMEM((CH,), jnp.int32),
        pltpu.VMEM((CH,), jnp.int32),
        pltpu.VMEM((CH,), jnp.int32),
        pltpu.VMEM((CH, C), jnp.float32),
        pltpu.VMEM((CH, C), jnp.float32),
        pltpu.VMEM_SHARED((NP, C), jnp.float32),
        pltpu.SemaphoreType.DMA,
        pltpu.SemaphoreType.DMA,
        pltpu.SemaphoreType.DMA,
        pltpu.SemaphoreType.DMA,
        pltpu.SemaphoreType.DMA,
        pltpu.SemaphoreType.DMA,
    ],
)(_seg_nocnt_body)


def _combine_body(relu, parts_ref, cnt_ref, x_ref, wl_ref, b_ref, wr_ref, o_ref):
    agg = parts_ref[0] + parts_ref[1]
    cnt = cnt_ref[0] + cnt_ref[1]
    mean = agg / jnp.maximum(cnt, 1.0)
    out = (
        lax.dot_general(mean, wl_ref[...], (((1,), (1,)), ((), ())),
                        preferred_element_type=jnp.float32)
        + lax.dot_general(x_ref[...], wr_ref[...], (((1,), (1,)), ((), ())),
                          preferred_element_type=jnp.float32)
        + b_ref[0:1, :]
    )
    if relu:
        out = jnp.maximum(out, 0.0)
    o_ref[...] = out


def _combine(parts, cnt1, x, w_l, b, w_r, relu):
    b8 = jnp.broadcast_to(b.reshape(1, C), (8, C))
    return pl.pallas_call(
        functools.partial(_combine_body, relu),
        grid=(N // BLK,),
        in_specs=[
            pl.BlockSpec((NC, BLK, C), lambda i: (0, i, 0)),
            pl.BlockSpec((NC, BLK, 1), lambda i: (0, i, 0)),
            pl.BlockSpec((BLK, C), lambda i: (i, 0)),
            pl.BlockSpec((C, C), lambda i: (0, 0)),
            pl.BlockSpec((8, C), lambda i: (0, 0)),
            pl.BlockSpec((C, C), lambda i: (0, 0)),
        ],
        out_specs=pl.BlockSpec((BLK, C), lambda i: (i, 0)),
        out_shape=jax.ShapeDtypeStruct((N, C), jnp.float32),
    )(parts, cnt1, x, w_l, b8, w_r)


def kernel(x, edge_index, W1_l, b1, W1_r, W2_l, b2, W2_r):
    ei = edge_index.astype(jnp.int32)
    src, dst = ei[0], ei[1]
    zacc = jnp.zeros((CH, C), jnp.float32)
    zcnt = jnp.zeros((RPT,), jnp.float32)
    ones = jnp.ones((CH,), jnp.float32)

    parts1, cntp = _seg_sum_cnt(x, src, dst, zacc, zcnt, ones)
    cnt1 = cntp.reshape(NC, NP, 1)
    h = _combine(parts1, cnt1, x, W1_l, b1, W1_r, relu=True)
    parts2 = _seg_sum(h, src, dst, zacc)
    out = _combine(parts2, cnt1, h, W2_l, b2, W2_r, relu=False)
    return out


# speedup vs baseline: 10.1284x; 1.0140x over previous
import functools

import jax
import jax.numpy as jnp
from jax import lax
from jax.experimental import pallas as pl
from jax.experimental.pallas import tpu as pltpu
from jax.experimental.pallas import tpu_sc as plsc

N = 10000
C = 128
E = 320000
NC = 2
NS = 16
NW = NC * NS
EPW = E // NW
CH = 80
NCHUNK = EPW // CH
NPAIR = (NCHUNK - 1) // 2
NP = 10240
RPT = NP // NS
BLK = 400

_mesh = plsc.VectorSubcoreMesh(
    core_axis_name="c", subcore_axis_name="s", num_cores=NC, num_subcores=NS
)


def _seg_body(with_cnt, x_hbm, src_hbm, dst_hbm, zacc_hbm, zcnt_hbm, ones_hbm,
              out_hbm, cnt_hbm, sidx0, sidx1, didx0, didx1, rows0, rows1,
              ones_v, cnt_v, acc_sh, cnt_sh, gs0, gs1, is0, is1, id0, id1,
              osem):
    cid = lax.axis_index("c")
    sid = lax.axis_index("s")
    sidx = (sidx0, sidx1)
    didx = (didx0, didx1)
    rows = (rows0, rows1)
    gs = (gs0, gs1)
    iss = (is0, is1)
    ids = (id0, id1)

    pltpu.sync_copy(zacc_hbm, rows0)
    if with_cnt:
        pltpu.sync_copy(ones_hbm, ones_v)
        pltpu.sync_copy(zcnt_hbm, cnt_v)
        pltpu.sync_copy(cnt_v, cnt_sh.at[pl.ds(sid * RPT, RPT)])
    for j in range(RPT // CH):
        pltpu.async_copy(rows0, acc_sh.at[pl.ds(sid * RPT + j * CH, CH)], gs0)
    for j in range(RPT // CH):
        pltpu.make_async_copy(
            rows0, acc_sh.at[pl.ds(sid * RPT + j * CH, CH)], gs0).wait()
    plsc.subcore_barrier()

    ebase = (cid * NS + sid) * EPW

    def off(c):
        return ebase + jnp.minimum(c, NCHUNK - 1) * CH

    def idx_load(c, b):
        pltpu.async_copy(src_hbm.at[pl.ds(off(c), CH)], sidx[b], iss[b])
        pltpu.async_copy(dst_hbm.at[pl.ds(off(c), CH)], didx[b], ids[b])

    def idx_wait(b):
        pltpu.make_async_copy(src_hbm.at[pl.ds(0, CH)], sidx[b], iss[b]).wait()
        pltpu.make_async_copy(dst_hbm.at[pl.ds(0, CH)], didx[b], ids[b]).wait()

    def gather(b):
        pltpu.async_copy(x_hbm.at[sidx[b]], rows[b], gs[b])

    def gather_wait(b):
        pltpu.make_async_copy(x_hbm.at[sidx[b]], rows[b], gs[b]).wait()

    def scatter(b):
        if with_cnt:
            pltpu.async_copy(ones_v, cnt_sh.at[didx[b]], osem, add=True)
        pltpu.sync_copy(rows[b], acc_sh.at[didx[b]], add=True)
        if with_cnt:
            pltpu.make_async_copy(ones_v, cnt_sh.at[didx[b]], osem).wait()

    pltpu.sync_copy(src_hbm.at[pl.ds(ebase, CH)], sidx0)
    pltpu.sync_copy(dst_hbm.at[pl.ds(ebase, CH)], didx0)
    idx_load(1, 1)
    gather(0)

    def body(i, carry):
        for b in (0, 1):
            c = 2 * i + b
            idx_wait(1 - b)
            gather(1 - b)
            gather_wait(b)
            scatter(b)
            idx_load(c + 2, b)
        return carry

    lax.fori_loop(0, NPAIR, body, 0)

    gather_wait(0)
    scatter(0)
    idx_wait(1)
    plsc.subcore_barrier()

    nw_ = RPT // CH
    bufs = (rows0, rows1)
    wsem = (gs0, gs1)

    def wo(j):
        return sid * RPT + j * CH

    pltpu.sync_copy(acc_sh.at[pl.ds(wo(0), CH)], rows0)
    for j in range(nw_):
        b = j % 2
        pltpu.async_copy(bufs[b], out_hbm.at[cid, pl.ds(wo(j), CH)], wsem[b])
        if j + 1 < nw_:
            if j >= 1:
                pltpu.make_async_copy(
                    bufs[1 - b], out_hbm.at[cid, pl.ds(wo(j - 1), CH)],
                    wsem[1 - b]).wait()
            pltpu.sync_copy(acc_sh.at[pl.ds(wo(j + 1), CH)], bufs[1 - b])
    pltpu.make_async_copy(
        bufs[nw_ % 2], out_hbm.at[cid, pl.ds(wo(nw_ - 2), CH)],
        wsem[nw_ % 2]).wait()
    pltpu.make_async_copy(
        bufs[(nw_ - 1) % 2], out_hbm.at[cid, pl.ds(wo(nw_ - 1), CH)],
        wsem[(nw_ - 1) % 2]).wait()
    if with_cnt:
        pltpu.sync_copy(cnt_sh.at[pl.ds(sid * RPT, RPT)], cnt_v)
        pltpu.sync_copy(cnt_v, cnt_hbm.at[pl.ds(cid * NP + sid * RPT, RPT)])


_seg_sum_cnt = functools.partial(
    pl.kernel,
    out_type=[
        jax.ShapeDtypeStruct((NC, NP, C), jnp.float32),
        jax.ShapeDtypeStruct((NC * NP,), jnp.float32),
    ],
    mesh=_mesh,
    scratch_types=[
        pltpu.VMEM((CH,), jnp.int32),
        pltpu.VMEM((CH,), jnp.int32),
        pltpu.VMEM((CH,), jnp.int32),
        pltpu.VMEM((CH,), jnp.int32),
        pltpu.VMEM((CH, C), jnp.float32),
        pltpu.VMEM((CH, C), jnp.float32),
        pltpu.VMEM((CH,), jnp.float32),
        pltpu.VMEM((RPT,), jnp.float32),
        pltpu.VMEM_SHARED((NP, C), jnp.float32),
        pltpu.VMEM_SHARED((NP,), jnp.float32),
        pltpu.SemaphoreType.DMA,
        pltpu.SemaphoreType.DMA,
        pltpu.SemaphoreType.DMA,
        pltpu.SemaphoreType.DMA,
        pltpu.SemaphoreType.DMA,
        pltpu.SemaphoreType.DMA,
        pltpu.SemaphoreType.DMA,
    ],
)(functools.partial(_seg_body, True))


def _seg_nocnt_body(x_hbm, src_hbm, dst_hbm, zacc_hbm, out_hbm, sidx0, sidx1,
                    didx0, didx1, rows0, rows1, acc_sh, gs0, gs1, is0, is1,
                    id0, id1):
    _seg_body(False, x_hbm, src_hbm, dst_hbm, zacc_hbm, None, None, out_hbm,
              None, sidx0, sidx1, didx0, didx1, rows0, rows1, None, None,
              acc_sh, None, gs0, gs1, is0, is1, id0, id1, None)


_seg_sum = functools.partial(
    pl.kernel,
    out_type=jax.ShapeDtypeStruct((NC, NP, C), jnp.float32),
    mesh=_mesh,
    scratch_types=[
        pltpu.VMEM((CH,), jnp.int32),
        pltpu.VMEM((CH,), jnp.int32),
        pltpu.VMEM((CH,), jnp.int32),
        pltpu.VMEM((CH,), jnp.int32),
        pltpu.VMEM((CH, C), jnp.float32),
        pltpu.VMEM((CH, C), jnp.float32),
        pltpu.VMEM_SHARED((NP, C), jnp.float32),
        pltpu.SemaphoreType.DMA,
        pltpu.SemaphoreType.DMA,
        pltpu.SemaphoreType.DMA,
        pltpu.SemaphoreType.DMA,
        pltpu.SemaphoreType.DMA,
        pltpu.SemaphoreType.DMA,
    ],
)(_seg_nocnt_body)


def _combine_body(relu, parts_ref, cnt_ref, x_ref, wl_ref, b_ref, wr_ref, o_ref):
    agg = parts_ref[0] + parts_ref[1]
    cnt = cnt_ref[0] + cnt_ref[1]
    mean = agg / jnp.maximum(cnt, 1.0)
    out = (
        lax.dot_general(mean, wl_ref[...], (((1,), (1,)), ((), ())),
                        preferred_element_type=jnp.float32)
        + lax.dot_general(x_ref[...], wr_ref[...], (((1,), (1,)), ((), ())),
                          preferred_element_type=jnp.float32)
        + b_ref[0:1, :]
    )
    if relu:
        out = jnp.maximum(out, 0.0)
    o_ref[...] = out


def _combine(parts, cnt1, x, w_l, b, w_r, relu):
    b8 = jnp.broadcast_to(b.reshape(1, C), (8, C))
    return pl.pallas_call(
        functools.partial(_combine_body, relu),
        grid=(N // BLK,),
        in_specs=[
            pl.BlockSpec((NC, BLK, C), lambda i: (0, i, 0)),
            pl.BlockSpec((NC, BLK, 1), lambda i: (0, i, 0)),
            pl.BlockSpec((BLK, C), lambda i: (i, 0)),
            pl.BlockSpec((C, C), lambda i: (0, 0)),
            pl.BlockSpec((8, C), lambda i: (0, 0)),
            pl.BlockSpec((C, C), lambda i: (0, 0)),
        ],
        out_specs=pl.BlockSpec((BLK, C), lambda i: (i, 0)),
        out_shape=jax.ShapeDtypeStruct((N, C), jnp.float32),
    )(parts, cnt1, x, w_l, b8, w_r)


def kernel(x, edge_index, W1_l, b1, W1_r, W2_l, b2, W2_r):
    ei = edge_index.astype(jnp.int32)
    src, dst = ei[0], ei[1]
    zacc = jnp.zeros((CH, C), jnp.float32)
    zcnt = jnp.zeros((RPT,), jnp.float32)
    ones = jnp.ones((CH,), jnp.float32)

    parts1, cntp = _seg_sum_cnt(x, src, dst, zacc, zcnt, ones)
    cnt1 = cntp.reshape(NC, NP, 1)
    h = _combine(parts1, cnt1, x, W1_l, b1, W1_r, relu=True)
    parts2 = _seg_sum(h, src, dst, zacc)
    out = _combine(parts2, cnt1, h, W2_l, b2, W2_r, relu=False)
    return out
